# baseline (speedup 1.0000x reference)
import functools

import jax
import jax.numpy as jnp
from jax.experimental import pallas as pl
from jax.experimental.pallas import tpu as pltpu
from jax.experimental.pallas import tpu_sc as plsc

_PREC = jax.lax.Precision.DEFAULT
_INV_SQRT2 = 0.7071067811865476
_NEG = float("-inf")


def _mm(a, b):
    return jax.lax.dot_general(
        a, b, (((1,), (0,)), ((), ())),
        precision=_PREC, preferred_element_type=jnp.float32)


def _mmT(a, b):
    return jax.lax.dot_general(
        a, b, (((1,), (1,)), ((), ())),
        precision=_PREC, preferred_element_type=jnp.float32)


def _pb_body(fpw_ref, w_ref, out_ref):
    j = pl.program_id(0)

    @pl.when(j == 0)
    def _():
        out_ref[...] = jnp.zeros_like(out_ref)

    out_ref[...] += _mm(fpw_ref[...], w_ref[...])


def _stage1_body(hs_ref, fk_ref, pb_ref, w1_ref, b1_ref,
                 probs_ref, resid_ref, h_ref):
    hs = hs_ref[...]
    scores = _mmT(hs, fk_ref[...])
    mx = jnp.max(scores, axis=-1, keepdims=True)
    e = jnp.exp(scores - mx)
    probs = e / jnp.sum(e, axis=-1, keepdims=True)
    probs_ref[...] = probs
    resid = hs - _mm(probs, pb_ref[...])
    resid_ref[...] = resid
    pre = _mmT(resid, w1_ref[...]) + b1_ref[...]
    h_ref[...] = 0.5 * pre * (1.0 + jax.lax.erf(pre * _INV_SQRT2))


def _topk_body(h_ref, w2_ref, b2_ref,
               sp_ref, idx_ref, vals_ref, sl_ref, *, top_k):
    tblk = h_ref.shape[0]
    dsz = w2_ref.shape[0]
    c = _mmT(h_ref[...], w2_ref[...]) + b2_ref[...]

    lane = jax.lax.broadcasted_iota(jnp.int32, (tblk, dsz), 1)
    lane8 = jax.lax.broadcasted_iota(jnp.int32, (tblk, top_k), 1)
    work = c
    idx8 = jnp.zeros((tblk, top_k), jnp.int32)
    val8 = jnp.zeros((tblk, top_k), jnp.float32)
    for k in range(top_k):
        mxv = jnp.max(work, axis=-1, keepdims=True)
        ii = jnp.min(jnp.where(work == mxv, lane, dsz), axis=-1,
                     keepdims=True)
        work = jnp.where(lane == ii, _NEG, work)
        idx8 = jnp.where(lane8 == k, ii, idx8)
        val8 = jnp.where(lane8 == k, mxv, val8)
    sp_ref[...] = jnp.where(work == _NEG, c, 0.0)
    idx_ref[...] = idx8
    vals_ref[...] = val8
    sl_ref[...] = jnp.sum(jnp.abs(val8)).reshape(1, 1, 1)


_SC_NC = 2
_SC_NS = 16
_SC_L = 16


def _sc_loss_body(w_hbm, idx_hbm, vals_hbm, resid_hbm, out_hbm,
                  idx_c, vals_c, rows_v, resid_v, acc_v, sem,
                  *, tpw, ct, top_k, d):
    wid = (jax.lax.axis_index("s") * _SC_NC + jax.lax.axis_index("c"))
    base_pair = wid * (tpw * top_k)
    base_tok = wid * tpw
    nchunk = tpw // ct
    rpc = ct * top_k

    def chunk_body(g, acc):
        off_pair = base_pair + g * rpc
        pltpu.sync_copy(idx_hbm.at[pl.ds(off_pair, rpc)], idx_c)
        pltpu.sync_copy(vals_hbm.at[pl.ds(off_pair, rpc)], vals_c)
        pltpu.async_copy(w_hbm.at[idx_c], rows_v, sem).wait()
        pltpu.sync_copy(resid_hbm.at[pl.ds(base_tok + g * ct, ct)], resid_v)
        vv = vals_c[...]
        dnums = jax.lax.GatherDimensionNumbers(
            offset_dims=(), collapsed_slice_dims=(0,), start_index_map=(0,))
        for t in range(ct):
            vb = [jax.lax.gather(
                      vv,
                      jnp.full((_SC_L, 1), t * top_k + k, jnp.int32),
                      dnums, (1,),
                      mode=jax.lax.GatherScatterMode.PROMISE_IN_BOUNDS)
                  for k in range(top_k)]

            def ds_body(s, a):
                sl = pl.ds(s * _SC_L, _SC_L)
                off16 = vb[0] * rows_v[t * top_k, sl]
                for k in range(1, top_k):
                    off16 = off16 + vb[k] * rows_v[t * top_k + k, sl]
                dd = off16 - resid_v[t, sl]
                return a + dd * dd

            acc = jax.lax.fori_loop(0, d // _SC_L, ds_body, acc)
        return acc

    acc = jax.lax.fori_loop(0, nchunk, chunk_body,
                            jnp.zeros((_SC_L,), jnp.float32))
    acc_v[...] = acc
    pltpu.sync_copy(acc_v, out_hbm.at[pl.ds(wid * _SC_L, _SC_L)])


def kernel(hidden_states, W, family_keys, family_proj_w, W1, b1, W2, b2):
    B, T, D = hidden_states.shape
    dict_size, _ = W.shape
    nf = family_keys.shape[0]
    top_k = 8

    x = hidden_states.reshape(T, D)
    b1r = b1.reshape(1, D)
    b2r = b2.reshape(1, dict_size)

    npb = 4
    pbk = dict_size // npb
    pb = pl.pallas_call(
        _pb_body,
        grid=(npb,),
        in_specs=[
            pl.BlockSpec((nf, pbk), lambda j: (0, j)),
            pl.BlockSpec((pbk, D), lambda j: (j, 0)),
        ],
        out_specs=pl.BlockSpec((nf, D), lambda j: (0, 0)),
        out_shape=jax.ShapeDtypeStruct((nf, D), jnp.float32),
    )(family_proj_w, W)

    tblk = 256 if T % 256 == 0 else T
    nt = T // tblk

    probs, resid, h = pl.pallas_call(
        _stage1_body,
        grid=(nt,),
        in_specs=[
            pl.BlockSpec((tblk, D), lambda i: (i, 0)),
            pl.BlockSpec((nf, D), lambda i: (0, 0)),
            pl.BlockSpec((nf, D), lambda i: (0, 0)),
            pl.BlockSpec((D, D), lambda i: (0, 0)),
            pl.BlockSpec((1, D), lambda i: (0, 0)),
        ],
        out_specs=[
            pl.BlockSpec((tblk, nf), lambda i: (i, 0)),
            pl.BlockSpec((tblk, D), lambda i: (i, 0)),
            pl.BlockSpec((tblk, D), lambda i: (i, 0)),
        ],
        out_shape=[
            jax.ShapeDtypeStruct((T, nf), jnp.float32),
            jax.ShapeDtypeStruct((T, D), jnp.float32),
            jax.ShapeDtypeStruct((T, D), jnp.float32),
        ],
        compiler_params=pltpu.CompilerParams(
            dimension_semantics=("arbitrary",)),
    )(x, family_keys, pb, W1, b1r)

    body = functools.partial(_topk_body, top_k=top_k)
    sp, idx, vals, slp = pl.pallas_call(
        body,
        grid=(nt,),
        in_specs=[
            pl.BlockSpec((tblk, D), lambda i: (i, 0)),
            pl.BlockSpec((dict_size, D), lambda i: (0, 0)),
            pl.BlockSpec((1, dict_size), lambda i: (0, 0)),
        ],
        out_specs=[
            pl.BlockSpec((tblk, dict_size), lambda i: (i, 0)),
            pl.BlockSpec((tblk, top_k), lambda i: (i, 0)),
            pl.BlockSpec((tblk, top_k), lambda i: (i, 0)),
            pl.BlockSpec((1, 1, 1), lambda i: (i, 0, 0)),
        ],
        out_shape=[
            jax.ShapeDtypeStruct((T, dict_size), jnp.float32),
            jax.ShapeDtypeStruct((T, top_k), jnp.int32),
            jax.ShapeDtypeStruct((T, top_k), jnp.float32),
            jax.ShapeDtypeStruct((nt, 1, 1), jnp.float32),
        ],
        compiler_params=pltpu.CompilerParams(
            dimension_semantics=("arbitrary",)),
    )(h, W2, b2r)

    nw = _SC_NC * _SC_NS
    tpw = T // nw
    ct = 2
    sc_body = functools.partial(
        _sc_loss_body, tpw=tpw, ct=ct, top_k=top_k, d=D)
    lossp = pl.kernel(
        sc_body,
        out_type=jax.ShapeDtypeStruct((nw * _SC_L,), jnp.float32),
        mesh=plsc.VectorSubcoreMesh(core_axis_name="c", subcore_axis_name="s"),
        scratch_types=[
            pltpu.VMEM((ct * top_k,), jnp.int32),
            pltpu.VMEM((ct * top_k,), jnp.float32),
            pltpu.VMEM((ct * top_k, D), jnp.float32),
            pltpu.VMEM((ct, D), jnp.float32),
            pltpu.VMEM((_SC_L,), jnp.float32),
            pltpu.SemaphoreType.DMA,
        ],
    )(W, idx.reshape(T * top_k), vals.reshape(T * top_k), resid)

    recon_loss = jnp.sum(lossp) / (T * D)
    sparsity_loss = jnp.sum(slp) / (T * dict_size)
    return (recon_loss, sparsity_loss,
            sp.reshape(B, T, dict_size), probs.reshape(B, T, nf))


# speedup vs baseline: 1.3569x; 1.3569x over previous
import functools

import jax
import jax.numpy as jnp
from jax.experimental import pallas as pl
from jax.experimental.pallas import tpu as pltpu
from jax.experimental.pallas import tpu_sc as plsc

_PREC = jax.lax.Precision.DEFAULT
_INV_SQRT2 = 0.7071067811865476
_NEG = float("-inf")


def _mm(a, b):
    return jax.lax.dot_general(
        a, b, (((1,), (0,)), ((), ())),
        precision=_PREC, preferred_element_type=jnp.float32)


def _mmT(a, b):
    return jax.lax.dot_general(
        a, b, (((1,), (1,)), ((), ())),
        precision=_PREC, preferred_element_type=jnp.float32)


def _pb_body(fpw_ref, w_ref, out_ref):
    j = pl.program_id(0)

    @pl.when(j == 0)
    def _():
        out_ref[...] = jnp.zeros_like(out_ref)

    out_ref[...] += _mm(fpw_ref[...], w_ref[...])


def _stage1_body(hs_ref, fk_ref, pb_ref, w1_ref, b1_ref,
                 probs_ref, resid_ref, h_ref):
    hs = hs_ref[...]
    scores = _mmT(hs, fk_ref[...])
    mx = jnp.max(scores, axis=-1, keepdims=True)
    e = jnp.exp(scores - mx)
    probs = e / jnp.sum(e, axis=-1, keepdims=True)
    probs_ref[...] = probs
    resid = hs - _mm(probs, pb_ref[...])
    resid_ref[...] = resid
    pre = _mmT(resid, w1_ref[...]) + b1_ref[...]
    h_ref[...] = 0.5 * pre * (1.0 + jax.lax.erf(pre * _INV_SQRT2))


def _topk_body(h_ref, w2_ref, b2_ref,
               sp_ref, idx_ref, vals_ref, sl_ref, *, top_k):
    tblk = h_ref.shape[0]
    dsz = w2_ref.shape[0]
    c = _mmT(h_ref[...], w2_ref[...]) + b2_ref[...]

    lane = jax.lax.broadcasted_iota(jnp.int32, (tblk, dsz), 1)
    lane8 = jax.lax.broadcasted_iota(jnp.int32, (tblk, top_k), 1)
    work = c
    idx8 = jnp.zeros((tblk, top_k), jnp.int32)
    val8 = jnp.zeros((tblk, top_k), jnp.float32)
    for k in range(top_k):
        mxv = jnp.max(work, axis=-1, keepdims=True)
        ii = jnp.min(jnp.where(work == mxv, lane, dsz), axis=-1,
                     keepdims=True)
        work = jnp.where(lane == ii, _NEG, work)
        idx8 = jnp.where(lane8 == k, ii, idx8)
        val8 = jnp.where(lane8 == k, mxv, val8)
    sp_ref[...] = jnp.where(work == _NEG, c, 0.0)
    idx_ref[...] = idx8
    vals_ref[...] = val8
    sl_ref[...] = jnp.sum(jnp.abs(val8)).reshape(1, 1, 1)


_SC_NC = 2
_SC_NS = 16
_SC_L = 16


def _sc_loss_body(w_hbm, idx_hbm, vals_hbm, resid_hbm, out_hbm,
                  idxv, valsv, rows_v, resid_v, acc_v,
                  gsem0, gsem1, rsem0, rsem1,
                  *, tpw, ct, top_k, d):
    wid = (jax.lax.axis_index("s") * _SC_NC + jax.lax.axis_index("c"))
    base_pair = wid * (tpw * top_k)
    base_tok = wid * tpw
    nchunk = tpw // ct
    rpc = ct * top_k
    gsem = (gsem0, gsem1)
    rsem = (rsem0, rsem1)

    pltpu.sync_copy(idx_hbm.at[pl.ds(base_pair, tpw * top_k)], idxv)
    pltpu.sync_copy(vals_hbm.at[pl.ds(base_pair, tpw * top_k)], valsv)

    def rows_dma(b, g):
        return pltpu.make_async_copy(
            w_hbm.at[idxv.at[pl.ds(g * rpc, rpc)]], rows_v.at[b], gsem[b])

    def resid_dma(b, g):
        return pltpu.make_async_copy(
            resid_hbm.at[pl.ds(base_tok + g * ct, ct)], resid_v.at[b], rsem[b])

    def start(b, g):
        rows_dma(b, g).start()
        resid_dma(b, g).start()

    dnums = jax.lax.GatherDimensionNumbers(
        offset_dims=(), collapsed_slice_dims=(0,), start_index_map=(0,))

    def compute(b, g, acc):
        vv = valsv[pl.ds(g * rpc, rpc)]
        for t in range(ct):
            vb = [jax.lax.gather(
                      vv,
                      jnp.full((_SC_L, 1), t * top_k + k, jnp.int32),
                      dnums, (1,),
                      mode=jax.lax.GatherScatterMode.PROMISE_IN_BOUNDS)
                  for k in range(top_k)]

            def ds_body(s, a):
                sl = pl.ds(s * _SC_L, _SC_L)
                off16 = vb[0] * rows_v[b, t * top_k, sl]
                for k in range(1, top_k):
                    off16 = off16 + vb[k] * rows_v[b, t * top_k + k, sl]
                dd = off16 - resid_v[b, t, sl]
                return a + dd * dd

            acc = jax.lax.fori_loop(0, d // _SC_L, ds_body, acc)
        return acc

    start(0, 0)
    start(1, 1)

    def pair_body(p, acc):
        for b in range(2):
            g = 2 * p + b
            rows_dma(b, g).wait()
            resid_dma(b, g).wait()
            acc = compute(b, g, acc)

            @pl.when(g + 2 < nchunk)
            def _():
                start(b, g + 2)
        return acc

    acc = jax.lax.fori_loop(0, nchunk // 2, pair_body,
                            jnp.zeros((_SC_L,), jnp.float32))
    acc_v[...] = acc
    pltpu.sync_copy(acc_v, out_hbm.at[pl.ds(wid * _SC_L, _SC_L)])


def kernel(hidden_states, W, family_keys, family_proj_w, W1, b1, W2, b2):
    B, T, D = hidden_states.shape
    dict_size, _ = W.shape
    nf = family_keys.shape[0]
    top_k = 8

    x = hidden_states.reshape(T, D)
    b1r = b1.reshape(1, D)
    b2r = b2.reshape(1, dict_size)

    npb = 4
    pbk = dict_size // npb
    pb = pl.pallas_call(
        _pb_body,
        grid=(npb,),
        in_specs=[
            pl.BlockSpec((nf, pbk), lambda j: (0, j)),
            pl.BlockSpec((pbk, D), lambda j: (j, 0)),
        ],
        out_specs=pl.BlockSpec((nf, D), lambda j: (0, 0)),
        out_shape=jax.ShapeDtypeStruct((nf, D), jnp.float32),
    )(family_proj_w, W)

    tblk = 256 if T % 256 == 0 else T
    nt = T // tblk

    probs, resid, h = pl.pallas_call(
        _stage1_body,
        grid=(nt,),
        in_specs=[
            pl.BlockSpec((tblk, D), lambda i: (i, 0)),
            pl.BlockSpec((nf, D), lambda i: (0, 0)),
            pl.BlockSpec((nf, D), lambda i: (0, 0)),
            pl.BlockSpec((D, D), lambda i: (0, 0)),
            pl.BlockSpec((1, D), lambda i: (0, 0)),
        ],
        out_specs=[
            pl.BlockSpec((tblk, nf), lambda i: (i, 0)),
            pl.BlockSpec((tblk, D), lambda i: (i, 0)),
            pl.BlockSpec((tblk, D), lambda i: (i, 0)),
        ],
        out_shape=[
            jax.ShapeDtypeStruct((T, nf), jnp.float32),
            jax.ShapeDtypeStruct((T, D), jnp.float32),
            jax.ShapeDtypeStruct((T, D), jnp.float32),
        ],
        compiler_params=pltpu.CompilerParams(
            dimension_semantics=("arbitrary",)),
    )(x, family_keys, pb, W1, b1r)

    body = functools.partial(_topk_body, top_k=top_k)
    sp, idx, vals, slp = pl.pallas_call(
        body,
        grid=(nt,),
        in_specs=[
            pl.BlockSpec((tblk, D), lambda i: (i, 0)),
            pl.BlockSpec((dict_size, D), lambda i: (0, 0)),
            pl.BlockSpec((1, dict_size), lambda i: (0, 0)),
        ],
        out_specs=[
            pl.BlockSpec((tblk, dict_size), lambda i: (i, 0)),
            pl.BlockSpec((tblk, top_k), lambda i: (i, 0)),
            pl.BlockSpec((tblk, top_k), lambda i: (i, 0)),
            pl.BlockSpec((1, 1, 1), lambda i: (i, 0, 0)),
        ],
        out_shape=[
            jax.ShapeDtypeStruct((T, dict_size), jnp.float32),
            jax.ShapeDtypeStruct((T, top_k), jnp.int32),
            jax.ShapeDtypeStruct((T, top_k), jnp.float32),
            jax.ShapeDtypeStruct((nt, 1, 1), jnp.float32),
        ],
        compiler_params=pltpu.CompilerParams(
            dimension_semantics=("arbitrary",)),
    )(h, W2, b2r)

    nw = _SC_NC * _SC_NS
    tpw = T // nw
    ct = 2
    sc_body = functools.partial(
        _sc_loss_body, tpw=tpw, ct=ct, top_k=top_k, d=D)
    lossp = pl.kernel(
        sc_body,
        out_type=jax.ShapeDtypeStruct((nw * _SC_L,), jnp.float32),
        mesh=plsc.VectorSubcoreMesh(core_axis_name="c", subcore_axis_name="s"),
        scratch_types=[
            pltpu.VMEM((tpw * top_k,), jnp.int32),
            pltpu.VMEM((tpw * top_k,), jnp.float32),
            pltpu.VMEM((2, ct * top_k, D), jnp.float32),
            pltpu.VMEM((2, ct, D), jnp.float32),
            pltpu.VMEM((_SC_L,), jnp.float32),
            pltpu.SemaphoreType.DMA,
            pltpu.SemaphoreType.DMA,
            pltpu.SemaphoreType.DMA,
            pltpu.SemaphoreType.DMA,
        ],
    )(W, idx.reshape(T * top_k), vals.reshape(T * top_k), resid)

    recon_loss = jnp.sum(lossp) / (T * D)
    sparsity_loss = jnp.sum(slp) / (T * dict_size)
    return (recon_loss, sparsity_loss,
            sp.reshape(B, T, dict_size), probs.reshape(B, T, nf))
